# Initial kernel scaffold; baseline (speedup 1.0000x reference)
import jax
import jax.numpy as jnp
from jax.experimental import pallas as pl


def kernel(position_ids, pe):
    raise NotImplementedError("write your pallas kernel here")


# speedup vs baseline: 2.4236x; 2.4236x over previous
import functools

import jax
import jax.numpy as jnp
from jax import lax
from jax.experimental import pallas as pl
from jax.experimental.pallas import tpu as pltpu
from jax.experimental.pallas import tpu_sc as plsc

D_MODEL = 768
B_TOTAL = 4 * 8192
NUM_CORES = 2
NUM_SUBCORES = 16
NUM_WORKERS = NUM_CORES * NUM_SUBCORES
B_PER_WORKER = B_TOTAL // NUM_WORKERS
CHUNK = 128
NUM_CHUNKS = B_PER_WORKER // CHUNK

_mesh = plsc.VectorSubcoreMesh(core_axis_name="c", subcore_axis_name="s")


@jax.jit
def _sc_gather(pe, idx_flat):
    @functools.partial(
        pl.kernel,
        mesh=_mesh,
        out_type=jax.ShapeDtypeStruct((B_TOTAL, D_MODEL), jnp.float32),
        scratch_types=[
            pltpu.VMEM((CHUNK,), jnp.int32),
            pltpu.VMEM((CHUNK, D_MODEL), jnp.float32),
            pltpu.SemaphoreType.DMA,
        ],
    )
    def k(table_hbm, idx_hbm, out_hbm, idx_v, rows_v, sem):
        wid = lax.axis_index("s") * NUM_CORES + lax.axis_index("c")
        base = wid * B_PER_WORKER

        @pl.loop(0, NUM_CHUNKS)
        def _(c):
            off = base + c * CHUNK
            pltpu.sync_copy(idx_hbm.at[pl.ds(off, CHUNK)], idx_v)
            pltpu.async_copy(table_hbm.at[idx_v], rows_v, sem).wait()
            pltpu.sync_copy(rows_v, out_hbm.at[pl.ds(off, CHUNK)])

    return k(pe, idx_flat)


def kernel(position_ids, pe):
    idx_flat = position_ids.reshape(-1).astype(jnp.int32)
    out = _sc_gather(pe, idx_flat)
    return out.reshape(position_ids.shape + (pe.shape[1],))


# speedup vs baseline: 2.4609x; 1.0154x over previous
import functools

import jax
import jax.numpy as jnp
from jax import lax
from jax.experimental import pallas as pl
from jax.experimental.pallas import tpu as pltpu
from jax.experimental.pallas import tpu_sc as plsc

D_MODEL = 768
B_TOTAL = 4 * 8192
NUM_CORES = 2
NUM_SUBCORES = 16
NUM_WORKERS = NUM_CORES * NUM_SUBCORES
B_PER_WORKER = B_TOTAL // NUM_WORKERS
CHUNK = 64
NUM_CHUNKS = B_PER_WORKER // CHUNK

_mesh = plsc.VectorSubcoreMesh(core_axis_name="c", subcore_axis_name="s")


@jax.jit
def _sc_gather(pe, idx_flat):
    @functools.partial(
        pl.kernel,
        mesh=_mesh,
        out_type=jax.ShapeDtypeStruct((B_TOTAL, D_MODEL), jnp.float32),
        scratch_types=[
            pltpu.VMEM((NUM_CHUNKS, CHUNK), jnp.int32),
            pltpu.VMEM((2, CHUNK, D_MODEL), jnp.float32),
            pltpu.SemaphoreType.DMA((2,)),
            pltpu.SemaphoreType.DMA((2,)),
        ],
    )
    def k(table_hbm, idx_hbm, out_hbm, idx_v, rows_v, gsem, ssem):
        wid = lax.axis_index("s") * NUM_CORES + lax.axis_index("c")
        base = wid * B_PER_WORKER
        pltpu.sync_copy(
            idx_hbm.at[pl.ds(wid * NUM_CHUNKS, NUM_CHUNKS)], idx_v
        )

        def start_gather(b, c):
            return pltpu.async_copy(
                table_hbm.at[idx_v.at[c]], rows_v.at[b], gsem.at[b]
            )

        def start_store(b, c):
            return pltpu.async_copy(
                rows_v.at[b], out_hbm.at[pl.ds(base + c * CHUNK, CHUNK)],
                ssem.at[b],
            )

        g = [None, None]
        s = [None, None]
        g[0] = start_gather(0, 0)
        for c in range(NUM_CHUNKS):
            b = c & 1
            nb = 1 - b
            if c + 1 < NUM_CHUNKS:
                if s[nb] is not None:
                    s[nb].wait()
                g[nb] = start_gather(nb, c + 1)
            g[b].wait()
            s[b] = start_store(b, c)
        s[0].wait()
        s[1].wait()

    return k(pe, idx_flat)


def kernel(position_ids, pe):
    idx_2d = position_ids.reshape(B_TOTAL // CHUNK, CHUNK).astype(jnp.int32)
    out = _sc_gather(pe, idx_2d)
    return out.reshape(position_ids.shape + (pe.shape[1],))
